# baseline (speedup 1.0000x reference)
import functools

import jax
import jax.numpy as jnp
from jax import lax
from jax.experimental import pallas as pl
from jax.experimental.pallas import tpu as pltpu
from jax.experimental.pallas import tpu_sc as plsc

VOCAB1 = 100001
D = 128
S = 50
NB = 4096
NC, NS = 2, 16
NW = NC * NS
C = 128
NCHUNK = S
NBUF = 6
GDEPTH = 4
FULL_GROUPS = NCHUNK // NBUF
TAIL = NCHUNK - FULL_GROUPS * NBUF


def _make_gather():
    mesh = plsc.VectorSubcoreMesh(core_axis_name="c", subcore_axis_name="s")

    @functools.partial(
        pl.kernel,
        mesh=mesh,
        out_type=jax.ShapeDtypeStruct((S, NB, D), jnp.float32),
        scratch_types=(
            [pltpu.VMEM((NCHUNK, C), jnp.int32)]
            + [pltpu.VMEM((C, D), jnp.float32) for _ in range(NBUF)]
            + [pltpu.SemaphoreType.DMA for _ in range(2 * NBUF)]
        ),
    )
    def gather(x_hbm, table_hbm, out_hbm, idx_v, *bufs_and_sems):
        rows = bufs_and_sems[:NBUF]
        sem_g = bufs_and_sems[NBUF:2 * NBUF]
        sem_o = bufs_and_sems[2 * NBUF:]
        wid = lax.axis_index("s") * NC + lax.axis_index("c")
        bbase = wid * C
        pltpu.sync_copy(x_hbm.at[:, pl.ds(bbase, C)], idx_v)

        def g_start(j, b):
            pltpu.async_copy(table_hbm.at[idx_v.at[j]], rows[b], sem_g[b])

        def g_wait(j, b):
            pltpu.make_async_copy(
                table_hbm.at[idx_v.at[j]], rows[b], sem_g[b]).wait()

        def o_start(j, b):
            pltpu.async_copy(
                rows[b], out_hbm.at[j, pl.ds(bbase, C)], sem_o[b])

        def o_wait(j, b):
            pltpu.make_async_copy(
                rows[b], out_hbm.at[j, pl.ds(bbase, C)], sem_o[b]).wait()

        def step(j, b):
            if isinstance(j, int):
                if j - (NBUF - GDEPTH) >= 0:
                    o_wait(j - (NBUF - GDEPTH), (b + GDEPTH) % NBUF)
                if j + GDEPTH < NCHUNK:
                    g_start(j + GDEPTH, (b + GDEPTH) % NBUF)
            else:
                o_wait(j - (NBUF - GDEPTH), (b + GDEPTH) % NBUF)
                g_start(j + GDEPTH, (b + GDEPTH) % NBUF)
            g_wait(j, b)
            o_start(j, b)

        for b in range(GDEPTH):
            g_start(b, b)
        for b in range(NBUF):
            step(b, b)

        def body(g, carry):
            j0 = g * NBUF
            for b in range(NBUF):
                step(j0 + b, b)
            return carry

        lax.fori_loop(1, FULL_GROUPS - 1, body, 0, unroll=False)

        for t in range((FULL_GROUPS - 1) * NBUF, NCHUNK):
            step(t, t % NBUF)
        for j in range(NCHUNK - (NBUF - GDEPTH), NCHUNK):
            o_wait(j, j % NBUF)

    return gather


_gather = _make_gather()


def kernel(x, table):
    out_sw = _gather(x.T, table)
    return out_sw.transpose(1, 0, 2)
